# baseline (speedup 1.0000x reference)
import jax
import jax.numpy as jnp
from jax.experimental import pallas as pl
from jax.experimental.pallas import tpu as pltpu

B = 1024
D_IN = 1024
H = 4096
D_OUT = 1024
BS = 64
N_BROW = H // BS
BLOCKS_PER_ROW = 16
FC1_TILES = 8
FC1_TILE = H // FC1_TILES
GRID = FC1_TILES + 1
CHUNK = 4
N_CHUNKS = BLOCKS_PER_ROW // CHUNK


def _mlp_kernel(cols_ref, w1_ref, x_ref, b1_ref, vt_ref, b2_ref,
                w3_ref, b3_ref, out_ref, h1t_ref, h2t_ref, xbf_ref):
    t = pl.program_id(0)

    @pl.when(t == 0)
    def _cast_x():
        xbf_ref[:] = x_ref[:].astype(jnp.bfloat16)

    @pl.when(t < FC1_TILES)
    def _fc1():
        acc = jax.lax.dot_general(
            w1_ref[:].astype(jnp.bfloat16), xbf_ref[:],
            (((1,), (1,)), ((), ())), preferred_element_type=jnp.float32)
        h1t_ref[pl.ds(t * FC1_TILE, FC1_TILE), :] = jnp.maximum(
            acc + b1_ref[:], 0.0).astype(jnp.bfloat16)

    @pl.when(t == FC1_TILES)
    def _bsr_fc3():
        def row(j, carry):
            base = j * BLOCKS_PER_ROW
            vj = vt_ref[j]
            partials = []
            for c in range(N_CHUNKS):
                parts = []
                for k in range(CHUNK * c, CHUNK * (c + 1)):
                    col = cols_ref[base + k]
                    parts.append(
                        h1t_ref[pl.ds(pl.multiple_of(col * BS, BS), BS), :])
                gt = jnp.concatenate(parts, axis=0)
                vc = vj[:, CHUNK * BS * c:CHUNK * BS * (c + 1)]
                partials.append(jax.lax.dot_general(
                    vc, gt, (((1,), (0,)), ((), ())),
                    preferred_element_type=jnp.float32))
            acc = (partials[0] + partials[1]) + (partials[2] + partials[3])
            b2j = b2_ref[pl.ds(j * BS, BS), :]
            h2t_ref[pl.ds(j * BS, BS), :] = jnp.maximum(
                acc + b2j, 0.0).astype(jnp.bfloat16)
            return carry

        jax.lax.fori_loop(0, N_BROW, row, 0)
        out_ref[:] = jax.lax.dot_general(
            h2t_ref[:], w3_ref[:], (((0,), (1,)), ((), ())),
            preferred_element_type=jnp.float32) + b3_ref[:]


def kernel(x, W1, b1, values, b2, W3, b3, crow_indices, col_indices):
    del crow_indices
    vt = values.reshape(N_BROW, BLOCKS_PER_ROW, BS, BS).transpose(
        0, 2, 1, 3).reshape(N_BROW, BS, BLOCKS_PER_ROW * BS).astype(jnp.bfloat16)
    w3_bf = W3.astype(jnp.bfloat16)
    b1c = b1.reshape(H, 1)
    b2c = b2.reshape(H, 1)
    b3r = b3.reshape(1, D_OUT)

    def _clamp_fc1(t, cols):
        return (jnp.minimum(t, FC1_TILES - 1), 0)

    grid_spec = pltpu.PrefetchScalarGridSpec(
        num_scalar_prefetch=1,
        grid=(GRID,),
        in_specs=[
            pl.BlockSpec((FC1_TILE, D_IN), _clamp_fc1),
            pl.BlockSpec((B, D_IN), lambda t, cols: (0, 0)),
            pl.BlockSpec((FC1_TILE, 1), _clamp_fc1),
            pl.BlockSpec((N_BROW, BS, BLOCKS_PER_ROW * BS),
                         lambda t, cols: (0, 0, 0)),
            pl.BlockSpec((H, 1), lambda t, cols: (0, 0)),
            pl.BlockSpec((D_OUT, H), lambda t, cols: (0, 0)),
            pl.BlockSpec((1, D_OUT), lambda t, cols: (0, 0)),
        ],
        out_specs=pl.BlockSpec((B, D_OUT), lambda t, cols: (0, 0)),
        scratch_shapes=[
            pltpu.VMEM((H, B), jnp.bfloat16),
            pltpu.VMEM((H, B), jnp.bfloat16),
            pltpu.VMEM((B, D_IN), jnp.bfloat16),
        ],
    )
    return pl.pallas_call(
        _mlp_kernel,
        grid_spec=grid_spec,
        out_shape=jax.ShapeDtypeStruct((B, D_OUT), jnp.float32),
    )(col_indices, W1, x, b1c, vt, b2c, w3_bf, b3r)


# speedup vs baseline: 1.2086x; 1.2086x over previous
import jax
import jax.numpy as jnp
from jax.experimental import pallas as pl
from jax.experimental.pallas import tpu as pltpu

B = 1024
D_IN = 1024
H = 4096
D_OUT = 1024
BS = 64
N_BROW = H // BS
BLOCKS_PER_ROW = 16
FC1_TILES = 8
FC1_TILE = H // FC1_TILES
CHUNK = 4
N_CHUNKS = BLOCKS_PER_ROW // CHUNK
FC3_TILES = 4
FC3_TILE = D_OUT // FC3_TILES


def _fc1_kernel(w1_ref, x_ref, b1_ref, h1t_ref, xbf_ref):
    t = pl.program_id(0)

    @pl.when(t == 0)
    def _cast_x():
        xbf_ref[:] = x_ref[:].astype(jnp.bfloat16)

    acc = jax.lax.dot_general(
        w1_ref[:].astype(jnp.bfloat16), xbf_ref[:],
        (((1,), (1,)), ((), ())), preferred_element_type=jnp.float32)
    h1t_ref[:] = jnp.maximum(acc + b1_ref[:], 0.0).astype(jnp.bfloat16)


def _bsr_kernel(cols_ref, h1t_ref, vals_ref, b2_ref, h2t_ref):
    def row(j, carry):
        base = j * BLOCKS_PER_ROW
        partials = []
        for c in range(N_CHUNKS):
            parts = []
            vparts = []
            for k in range(CHUNK * c, CHUNK * (c + 1)):
                col = cols_ref[base + k]
                parts.append(
                    h1t_ref[pl.ds(pl.multiple_of(col * BS, BS), BS), :])
                vparts.append(vals_ref[base + k])
            gt = jnp.concatenate(parts, axis=0)
            vc = jnp.concatenate(vparts, axis=1).astype(jnp.bfloat16)
            partials.append(jax.lax.dot_general(
                vc, gt, (((1,), (0,)), ((), ())),
                preferred_element_type=jnp.float32))
        acc = (partials[0] + partials[1]) + (partials[2] + partials[3])
        b2j = b2_ref[pl.ds(j * BS, BS), :]
        h2t_ref[pl.ds(j * BS, BS), :] = jnp.maximum(
            acc + b2j, 0.0).astype(jnp.bfloat16)
        return carry

    jax.lax.fori_loop(0, N_BROW, row, 0)


def _fc3_kernel(h2t_ref, w3_ref, b3_ref, out_ref):
    out_ref[:] = jax.lax.dot_general(
        h2t_ref[:], w3_ref[:].astype(jnp.bfloat16),
        (((0,), (1,)), ((), ())),
        preferred_element_type=jnp.float32) + b3_ref[:]


def kernel(x, W1, b1, values, b2, W3, b3, crow_indices, col_indices):
    del crow_indices
    b1c = b1.reshape(H, 1)
    b2c = b2.reshape(H, 1)
    b3r = b3.reshape(1, D_OUT)

    h1t = pl.pallas_call(
        _fc1_kernel,
        grid=(FC1_TILES,),
        in_specs=[
            pl.BlockSpec((FC1_TILE, D_IN), lambda t: (t, 0)),
            pl.BlockSpec((B, D_IN), lambda t: (0, 0)),
            pl.BlockSpec((FC1_TILE, 1), lambda t: (t, 0)),
        ],
        out_specs=pl.BlockSpec((FC1_TILE, B), lambda t: (t, 0)),
        out_shape=jax.ShapeDtypeStruct((H, B), jnp.bfloat16),
        scratch_shapes=[pltpu.VMEM((B, D_IN), jnp.bfloat16)],
    )(W1, x, b1c)

    grid_spec = pltpu.PrefetchScalarGridSpec(
        num_scalar_prefetch=1,
        grid=(1,),
        in_specs=[
            pl.BlockSpec((H, B), lambda t, cols: (0, 0)),
            pl.BlockSpec((N_BROW * BLOCKS_PER_ROW, BS, BS),
                         lambda t, cols: (0, 0, 0)),
            pl.BlockSpec((H, 1), lambda t, cols: (0, 0)),
        ],
        out_specs=pl.BlockSpec((H, B), lambda t, cols: (0, 0)),
    )
    h2t = pl.pallas_call(
        _bsr_kernel,
        grid_spec=grid_spec,
        out_shape=jax.ShapeDtypeStruct((H, B), jnp.bfloat16),
    )(col_indices, h1t, values, b2c)

    return pl.pallas_call(
        _fc3_kernel,
        grid=(FC3_TILES,),
        in_specs=[
            pl.BlockSpec((H, B), lambda t: (0, 0)),
            pl.BlockSpec((FC3_TILE, H), lambda t: (t, 0)),
            pl.BlockSpec((1, FC3_TILE), lambda t: (0, t)),
        ],
        out_specs=pl.BlockSpec((B, FC3_TILE), lambda t: (0, t)),
        out_shape=jax.ShapeDtypeStruct((B, D_OUT), jnp.float32),
    )(h2t, W3, b3r)


# speedup vs baseline: 1.2374x; 1.0238x over previous
import jax
import jax.numpy as jnp
from jax.experimental import pallas as pl
from jax.experimental.pallas import tpu as pltpu

B = 1024
D_IN = 1024
H = 4096
D_OUT = 1024
BS = 64
N_BROW = H // BS
BLOCKS_PER_ROW = 16
NNZ = N_BROW * BLOCKS_PER_ROW
FC1_TILES = 16
FC1_TILE = H // FC1_TILES
CHUNK = 4
N_CHUNKS = BLOCKS_PER_ROW // CHUNK
FC3_TILES = 8
FC3_TILE = D_OUT // FC3_TILES
BSR_STEP = FC1_TILES
GRID = FC1_TILES + 1 + FC3_TILES


def _mlp_kernel(cols_ref, w1_ref, xbf_ref, b1_ref, vals_ref, b2_ref,
                w3_ref, b3_ref, out_ref, h1t_ref, h2t_ref):
    t = pl.program_id(0)

    @pl.when(t < FC1_TILES)
    def _fc1():
        acc = jax.lax.dot_general(
            w1_ref[:].astype(jnp.bfloat16), xbf_ref[:],
            (((1,), (1,)), ((), ())), preferred_element_type=jnp.float32)
        h1t_ref[pl.ds(t * FC1_TILE, FC1_TILE), :] = jnp.maximum(
            acc + b1_ref[:], 0.0).astype(jnp.bfloat16)

    @pl.when(t == BSR_STEP)
    def _bsr():
        def row(j, carry):
            base = j * BLOCKS_PER_ROW
            partials = []
            for c in range(N_CHUNKS):
                parts = []
                vparts = []
                for k in range(CHUNK * c, CHUNK * (c + 1)):
                    col = cols_ref[base + k]
                    parts.append(
                        h1t_ref[pl.ds(pl.multiple_of(col * BS, BS), BS), :])
                    vparts.append(vals_ref[base + k])
                gt = jnp.concatenate(parts, axis=0)
                vc = jnp.concatenate(vparts, axis=1)
                partials.append(jax.lax.dot_general(
                    vc, gt, (((1,), (0,)), ((), ())),
                    preferred_element_type=jnp.float32))
            acc = (partials[0] + partials[1]) + (partials[2] + partials[3])
            b2j = b2_ref[pl.ds(j * BS, BS), :]
            h2t_ref[pl.ds(j * BS, BS), :] = jnp.maximum(
                acc + b2j, 0.0).astype(jnp.bfloat16)
            return carry

        jax.lax.fori_loop(0, N_BROW, row, 0)

    @pl.when(t > BSR_STEP)
    def _fc3():
        out_ref[:] = jax.lax.dot_general(
            h2t_ref[:], w3_ref[:].astype(jnp.bfloat16),
            (((0,), (1,)), ((), ())),
            preferred_element_type=jnp.float32) + b3_ref[:]


def kernel(x, W1, b1, values, b2, W3, b3, crow_indices, col_indices):
    del crow_indices
    x_bf = x.astype(jnp.bfloat16)
    vals_bf = values.astype(jnp.bfloat16)
    b1c = b1.reshape(H, 1)
    b2c = b2.reshape(H, 1)
    b3r = b3.reshape(1, D_OUT)

    def _fc1_idx(t, cols):
        return (jnp.minimum(t, FC1_TILES - 1), 0)

    def _fc3_idx(t, cols):
        return (jnp.clip(t - BSR_STEP - 1, 0, FC3_TILES - 1), 0)

    def _fc3_bidx(t, cols):
        return (0, jnp.clip(t - BSR_STEP - 1, 0, FC3_TILES - 1))

    grid_spec = pltpu.PrefetchScalarGridSpec(
        num_scalar_prefetch=1,
        grid=(GRID,),
        in_specs=[
            pl.BlockSpec((FC1_TILE, D_IN), _fc1_idx),
            pl.BlockSpec((B, D_IN), lambda t, cols: (0, 0)),
            pl.BlockSpec((FC1_TILE, 1), _fc1_idx),
            pl.BlockSpec((NNZ, BS, BS), lambda t, cols: (0, 0, 0)),
            pl.BlockSpec((H, 1), lambda t, cols: (0, 0)),
            pl.BlockSpec((FC3_TILE, H), _fc3_idx),
            pl.BlockSpec((1, FC3_TILE), _fc3_bidx),
        ],
        out_specs=pl.BlockSpec((B, FC3_TILE), _fc3_bidx),
        scratch_shapes=[
            pltpu.VMEM((H, B), jnp.bfloat16),
            pltpu.VMEM((H, B), jnp.bfloat16),
        ],
    )
    return pl.pallas_call(
        _mlp_kernel,
        grid_spec=grid_spec,
        out_shape=jax.ShapeDtypeStruct((B, D_OUT), jnp.float32),
    )(col_indices, W1, x_bf, b1c, vals_bf, b2c, W3, b3r)


# speedup vs baseline: 1.2647x; 1.0221x over previous
import jax
import jax.numpy as jnp
from jax.experimental import pallas as pl
from jax.experimental.pallas import tpu as pltpu

B = 1024
D_IN = 1024
H = 4096
D_OUT = 1024
BS = 64
N_BROW = H // BS
BLOCKS_PER_ROW = 16
NNZ = N_BROW * BLOCKS_PER_ROW
FC1_TILES = 16
FC1_TILE = H // FC1_TILES
ROWS_PER_FC1 = N_BROW // FC1_TILES
CHUNK = 4
N_CHUNKS = BLOCKS_PER_ROW // CHUNK
FC3_TILES = 8
FC3_TILE = D_OUT // FC3_TILES
BSR_STEP = FC1_TILES
GRID = FC1_TILES + 1 + FC3_TILES


def _mlp_kernel(cols_ref, w1_ref, xbf_ref, b1_ref, vals_ref, b2_ref,
                w3_ref, b3_ref, out_ref, h1t_ref, h2t_ref, vt_ref):
    t = pl.program_id(0)

    @pl.when(t < FC1_TILES)
    def _fc1():
        acc = jax.lax.dot_general(
            w1_ref[:].astype(jnp.bfloat16), xbf_ref[:],
            (((1,), (1,)), ((), ())), preferred_element_type=jnp.float32)
        h1t_ref[pl.ds(t * FC1_TILE, FC1_TILE), :] = jnp.maximum(
            acc + b1_ref[:], 0.0).astype(jnp.bfloat16)
        for i in range(ROWS_PER_FC1):
            j = t * ROWS_PER_FC1 + i
            vt_ref[j] = jnp.concatenate(
                [vals_ref[i * BLOCKS_PER_ROW + k].astype(jnp.bfloat16)
                 for k in range(BLOCKS_PER_ROW)], axis=1)

    @pl.when(t == BSR_STEP)
    def _bsr():
        def row(j, carry):
            base = j * BLOCKS_PER_ROW
            vj = vt_ref[j]
            partials = []
            for c in range(N_CHUNKS):
                parts = []
                for k in range(CHUNK * c, CHUNK * (c + 1)):
                    col = cols_ref[base + k]
                    parts.append(
                        h1t_ref[pl.ds(pl.multiple_of(col * BS, BS), BS), :])
                gt = jnp.concatenate(parts, axis=0)
                vc = vj[:, CHUNK * BS * c:CHUNK * BS * (c + 1)]
                partials.append(jax.lax.dot_general(
                    vc, gt, (((1,), (0,)), ((), ())),
                    preferred_element_type=jnp.float32))
            acc = (partials[0] + partials[1]) + (partials[2] + partials[3])
            b2j = b2_ref[pl.ds(j * BS, BS), :]
            h2t_ref[pl.ds(j * BS, BS), :] = jnp.maximum(
                acc + b2j, 0.0).astype(jnp.bfloat16)
            return carry

        jax.lax.fori_loop(0, N_BROW, row, 0)

    @pl.when(t > BSR_STEP)
    def _fc3():
        out_ref[:] = jax.lax.dot_general(
            h2t_ref[:], w3_ref[:].astype(jnp.bfloat16),
            (((0,), (1,)), ((), ())),
            preferred_element_type=jnp.float32) + b3_ref[:]


def kernel(x, W1, b1, values, b2, W3, b3, crow_indices, col_indices):
    del crow_indices
    x_bf = x.astype(jnp.bfloat16)
    b1c = b1.reshape(H, 1)
    b2c = b2.reshape(H, 1)
    b3r = b3.reshape(1, D_OUT)

    def _fc1_idx(t, cols):
        return (jnp.minimum(t, FC1_TILES - 1), 0)

    def _fc3_idx(t, cols):
        return (jnp.clip(t - BSR_STEP - 1, 0, FC3_TILES - 1), 0)

    def _fc3_bidx(t, cols):
        return (0, jnp.clip(t - BSR_STEP - 1, 0, FC3_TILES - 1))

    grid_spec = pltpu.PrefetchScalarGridSpec(
        num_scalar_prefetch=1,
        grid=(GRID,),
        in_specs=[
            pl.BlockSpec((FC1_TILE, D_IN), _fc1_idx),
            pl.BlockSpec((B, D_IN), lambda t, cols: (0, 0)),
            pl.BlockSpec((FC1_TILE, 1), _fc1_idx),
            pl.BlockSpec((ROWS_PER_FC1 * BLOCKS_PER_ROW, BS, BS),
                         lambda t, cols: (jnp.minimum(t, FC1_TILES - 1), 0, 0)),
            pl.BlockSpec((H, 1), lambda t, cols: (0, 0)),
            pl.BlockSpec((FC3_TILE, H), _fc3_idx),
            pl.BlockSpec((1, FC3_TILE), _fc3_bidx),
        ],
        out_specs=pl.BlockSpec((B, FC3_TILE), _fc3_bidx),
        scratch_shapes=[
            pltpu.VMEM((H, B), jnp.bfloat16),
            pltpu.VMEM((H, B), jnp.bfloat16),
            pltpu.VMEM((N_BROW, BS, BLOCKS_PER_ROW * BS), jnp.bfloat16),
        ],
    )
    return pl.pallas_call(
        _mlp_kernel,
        grid_spec=grid_spec,
        out_shape=jax.ShapeDtypeStruct((B, D_OUT), jnp.float32),
    )(col_indices, W1, x_bf, b1c, values, b2c, W3, b3r)


# speedup vs baseline: 1.4186x; 1.1217x over previous
import jax
import jax.numpy as jnp
from jax.experimental import pallas as pl
from jax.experimental.pallas import tpu as pltpu

B = 1024
D_IN = 1024
H = 4096
D_OUT = 1024
BS = 64
N_BROW = H // BS
BLOCKS_PER_ROW = 16
NNZ = N_BROW * BLOCKS_PER_ROW
FC1_TILES = 8
FC1_TILE = H // FC1_TILES
ROWS_PER_FC1 = N_BROW // FC1_TILES
CHUNK = 8
N_CHUNKS = BLOCKS_PER_ROW // CHUNK
FC3_TILES = 4
FC3_TILE = D_OUT // FC3_TILES
BSR_STEP = FC1_TILES
GRID = FC1_TILES + 1 + FC3_TILES


def _mlp_kernel(cols_ref, w1_ref, xbf_ref, b1_ref, vals_ref, b2_ref,
                w3_ref, b3_ref, out_ref, h1t_ref, h2t_ref, vt_ref):
    t = pl.program_id(0)

    @pl.when(t < FC1_TILES)
    def _fc1():
        acc = jax.lax.dot_general(
            w1_ref[:].astype(jnp.bfloat16), xbf_ref[:],
            (((1,), (1,)), ((), ())), preferred_element_type=jnp.float32)
        h1t_ref[pl.ds(t * FC1_TILE, FC1_TILE), :] = jnp.maximum(
            acc + b1_ref[:], 0.0).astype(jnp.bfloat16)
        for i in range(ROWS_PER_FC1):
            j = t * ROWS_PER_FC1 + i
            vt_ref[j] = jnp.concatenate(
                [vals_ref[i * BLOCKS_PER_ROW + k].astype(jnp.bfloat16)
                 for k in range(BLOCKS_PER_ROW)], axis=1)

    @pl.when(t == BSR_STEP)
    def _bsr():
        def row(j, carry):
            base = j * BLOCKS_PER_ROW
            vj = vt_ref[j]
            partials = []
            for c in range(N_CHUNKS):
                parts = []
                for k in range(CHUNK * c, CHUNK * (c + 1)):
                    col = cols_ref[base + k]
                    parts.append(
                        h1t_ref[pl.ds(pl.multiple_of(col * BS, BS), BS), :])
                gt = jnp.concatenate(parts, axis=0)
                vc = vj[:, CHUNK * BS * c:CHUNK * BS * (c + 1)]
                partials.append(jax.lax.dot_general(
                    vc, gt, (((1,), (0,)), ((), ())),
                    preferred_element_type=jnp.float32))
            acc = sum(partials[1:], partials[0])
            b2j = b2_ref[pl.ds(j * BS, BS), :]
            h2t_ref[pl.ds(j * BS, BS), :] = jnp.maximum(
                acc + b2j, 0.0).astype(jnp.bfloat16)
            return carry

        jax.lax.fori_loop(0, N_BROW, row, 0)

    @pl.when(t > BSR_STEP)
    def _fc3():
        out_ref[:] = jax.lax.dot_general(
            h2t_ref[:], w3_ref[:].astype(jnp.bfloat16),
            (((0,), (1,)), ((), ())),
            preferred_element_type=jnp.float32) + b3_ref[:]


def kernel(x, W1, b1, values, b2, W3, b3, crow_indices, col_indices):
    del crow_indices
    x_bf = x.astype(jnp.bfloat16)
    b1c = b1.reshape(H, 1)
    b2c = b2.reshape(H, 1)
    b3r = b3.reshape(1, D_OUT)

    def _fc1_idx(t, cols):
        return (jnp.minimum(t, FC1_TILES - 1), 0)

    def _fc3_idx(t, cols):
        return (jnp.clip(t - BSR_STEP - 1, 0, FC3_TILES - 1), 0)

    def _fc3_bidx(t, cols):
        return (0, jnp.clip(t - BSR_STEP - 1, 0, FC3_TILES - 1))

    grid_spec = pltpu.PrefetchScalarGridSpec(
        num_scalar_prefetch=1,
        grid=(GRID,),
        in_specs=[
            pl.BlockSpec((FC1_TILE, D_IN), _fc1_idx),
            pl.BlockSpec((B, D_IN), lambda t, cols: (0, 0)),
            pl.BlockSpec((FC1_TILE, 1), _fc1_idx),
            pl.BlockSpec((ROWS_PER_FC1 * BLOCKS_PER_ROW, BS, BS),
                         lambda t, cols: (jnp.minimum(t, FC1_TILES - 1), 0, 0)),
            pl.BlockSpec((H, 1), lambda t, cols: (0, 0)),
            pl.BlockSpec((FC3_TILE, H), _fc3_idx),
            pl.BlockSpec((1, FC3_TILE), _fc3_bidx),
        ],
        out_specs=pl.BlockSpec((B, FC3_TILE), _fc3_bidx),
        scratch_shapes=[
            pltpu.VMEM((H, B), jnp.bfloat16),
            pltpu.VMEM((H, B), jnp.bfloat16),
            pltpu.VMEM((N_BROW, BS, BLOCKS_PER_ROW * BS), jnp.bfloat16),
        ],
    )
    return pl.pallas_call(
        _mlp_kernel,
        grid_spec=grid_spec,
        out_shape=jax.ShapeDtypeStruct((B, D_OUT), jnp.float32),
    )(col_indices, W1, x_bf, b1c, values, b2c, W3, b3r)
